# baseline (device time: 21507 ns/iter reference)
import jax
import jax.numpy as jnp
from jax import lax
from jax.experimental import pallas as pl
from jax.experimental.pallas import tpu as pltpu

N_DEV = 4

_CompilerParams = getattr(pltpu, "CompilerParams", None) or pltpu.TPUCompilerParams


def kernel(x, Wq, K_ext, V_ext, Wo):
    B, sq_loc, D = x.shape
    _, skv, hq, dh = K_ext.shape
    Kt = jnp.transpose(K_ext, (0, 2, 3, 1))
    Vt = jnp.transpose(V_ext, (0, 2, 3, 1))
    d_out = Wo.shape[1]
    rows = B * sq_loc
    h_grp = hq // N_DEV
    grp_cols = h_grp * dh
    bf16 = jnp.bfloat16
    wq_sh = Wq.shape

    def body(x_ref, wq_ref, k_ref, v_ref, wo_ref, out_ref,
             wqb, wob, pair_send, pair_l, pair_r, fwd_wq, fwd_wo,
             k_stage, v_stage, k_bf, v_bf, ctx_blk,
             k_sem, v_sem,
             s_pair_l, s_pair_r, r_pair_l, r_pair_r,
             s_fwdq, r_fwdq, s_fwdo, r_fwdo):
        my = lax.axis_index("i")
        left = lax.rem(my + N_DEV - 1, N_DEV)
        right = lax.rem(my + 1, N_DEV)
        is_even = lax.rem(my, 2) == 0
        MESH = pl.DeviceIdType.MESH

        kdma = pltpu.make_async_copy(k_ref, k_stage, k_sem)
        vdma = pltpu.make_async_copy(v_ref, v_stage, v_sem)
        kdma.start()
        vdma.start()

        wqb[...] = wq_ref[...].astype(bf16)
        wob[...] = wo_ref[...].astype(bf16)
        pair_send[0] = wqb[...]
        pair_send[1] = wob[...].reshape(wq_sh)
        x2b = x_ref[...].reshape(rows, D).astype(bf16)

        def stage_kv():
            kdma.wait()
            vdma.wait()
            for b in range(B):
                k_bf[:, :, b * skv:(b + 1) * skv] = k_stage[b].astype(bf16)
                v_bf[:, :, b * skv:(b + 1) * skv] = v_stage[b].astype(bf16)

        barrier_sem = pltpu.get_barrier_semaphore()
        for nbr in (left, right):
            pl.semaphore_signal(barrier_sem, inc=1, device_id=(nbr,),
                                device_id_type=MESH)
        pl.semaphore_wait(barrier_sem, 2)

        def copy(src, dst, ssem, rsem, dev):
            return pltpu.make_async_remote_copy(
                src_ref=src, dst_ref=dst, send_sem=ssem, recv_sem=rsem,
                device_id=(dev,), device_id_type=MESH)

        ri = lax.broadcasted_iota(jnp.int32, (rows, B * skv), 0)
        ci = lax.broadcasted_iota(jnp.int32, (rows, B * skv), 1)
        qb = my * (sq_loc // 64) + lax.rem(ri, sq_loc) // 64
        kb = lax.rem(ci, skv) // 64
        same_b = (ri // sq_loc) == (ci // skv)
        mask = same_b & ((qb == kb) | (lax.rem(kb, 4) == lax.rem(qb, 4)))
        row_keep = jnp.any(mask, axis=1, keepdims=True)
        neg = jnp.float32(-1e9)

        def group_out(wq_val, wo_val, origin):
            qg = (jnp.dot(x2b, wq_val, preferred_element_type=jnp.float32)
                  * 0.125).astype(bf16)
            for h2 in range(h_grp):
                c0 = h2 * dh
                g = origin * h_grp + h2
                kcat = k_bf[g]
                vcat = v_bf[g]
                qc = qg[:, c0:c0 + dh]
                s = jnp.dot(qc, kcat,
                            preferred_element_type=jnp.float32)
                s = jnp.where(mask, s, neg)
                w = jnp.exp(s)
                ws = jnp.where(row_keep,
                               jnp.sum(w, axis=1, keepdims=True), 1.0)
                w = jnp.where(row_keep, w / ws, 0.0)
                ctx_blk[:, c0:c0 + dh] = lax.dot_general(
                    w.astype(bf16), vcat, (((1,), (1,)), ((), ())),
                    preferred_element_type=jnp.float32).astype(bf16)
            return jnp.dot(ctx_blk[...], wo_val,
                           preferred_element_type=jnp.float32)

        @pl.when(is_even)
        def _():
            cq = copy(wqb, fwd_wq, s_fwdq, r_fwdq, left)
            co = copy(wob, fwd_wo, s_fwdo, r_fwdo, right)
            cq.start()
            co.start()

            stage_kv()

            acc = group_out(wqb[...], wob[...], my)

            copy(pair_send, pair_l, s_pair_l, r_pair_l, left).wait_recv()
            acc = acc + group_out(pair_l[0],
                                  pair_l[1].reshape(grp_cols, d_out),
                                  lax.rem(my + N_DEV - 1, N_DEV))
            copy(pair_send, pair_r, s_pair_r, r_pair_r, right).wait_recv()
            acc = acc + group_out(pair_r[0],
                                  pair_r[1].reshape(grp_cols, d_out),
                                  lax.rem(my + 1, N_DEV))

            cq.wait_recv()
            co.wait_recv()
            acc = acc + group_out(fwd_wq[...], fwd_wo[...],
                                  lax.rem(my + 2, N_DEV))

            cq.wait_send()
            co.wait_send()
            out_ref[...] = acc.reshape(B, sq_loc, d_out)

        @pl.when(jnp.logical_not(is_even))
        def _():
            cl = copy(pair_send, pair_r, s_pair_l, r_pair_r, left)
            cr = copy(pair_send, pair_l, s_pair_r, r_pair_l, right)
            cl.start()
            cr.start()

            kdma.wait()
            vdma.wait()

            copy(wqb, fwd_wq, s_fwdq, r_fwdq, right).wait_recv()
            fq = copy(fwd_wq, fwd_wq, s_fwdq, r_fwdq, left)
            fq.start()
            copy(wob, fwd_wo, s_fwdo, r_fwdo, left).wait_recv()
            fo = copy(fwd_wo, fwd_wo, s_fwdo, r_fwdo, right)
            fo.start()

            cl.wait_send()
            cr.wait_send()
            fq.wait_send()
            fo.wait_send()
            out_ref[...] = jnp.zeros((B, sq_loc, d_out), jnp.float32)

    return pl.pallas_call(
        body,
        out_shape=jax.ShapeDtypeStruct((B, sq_loc, d_out), jnp.float32),
        in_specs=[
            pl.BlockSpec(memory_space=pltpu.VMEM),
            pl.BlockSpec(memory_space=pltpu.VMEM),
            pl.BlockSpec(memory_space=pl.ANY),
            pl.BlockSpec(memory_space=pl.ANY),
            pl.BlockSpec(memory_space=pltpu.VMEM),
        ],
        out_specs=pl.BlockSpec(memory_space=pltpu.VMEM),
        scratch_shapes=[
            pltpu.VMEM(wq_sh, bf16),
            pltpu.VMEM(Wo.shape, bf16),
            pltpu.VMEM((2,) + wq_sh, bf16),
            pltpu.VMEM((2,) + wq_sh, bf16),
            pltpu.VMEM((2,) + wq_sh, bf16),
            pltpu.VMEM(wq_sh, bf16),
            pltpu.VMEM(Wo.shape, bf16),
            pltpu.VMEM(Kt.shape, Kt.dtype),
            pltpu.VMEM(Vt.shape, Vt.dtype),
            pltpu.VMEM((hq, dh, B * skv), bf16),
            pltpu.VMEM((hq, dh, B * skv), bf16),
            pltpu.VMEM((rows, grp_cols), bf16),
            pltpu.SemaphoreType.DMA,
            pltpu.SemaphoreType.DMA,
            pltpu.SemaphoreType.DMA,
            pltpu.SemaphoreType.DMA,
            pltpu.SemaphoreType.DMA,
            pltpu.SemaphoreType.DMA,
            pltpu.SemaphoreType.DMA,
            pltpu.SemaphoreType.DMA,
            pltpu.SemaphoreType.DMA,
            pltpu.SemaphoreType.DMA,
        ],
        compiler_params=_CompilerParams(collective_id=0),
    )(x, Wq, Kt, Vt, Wo)


# device time: 18039 ns/iter; 1.1923x vs baseline; 1.1923x over previous
import jax
import jax.numpy as jnp
from jax import lax
from jax.experimental import pallas as pl
from jax.experimental.pallas import tpu as pltpu

N_DEV = 4

_CompilerParams = getattr(pltpu, "CompilerParams", None) or pltpu.TPUCompilerParams


def kernel(x, Wq, K_ext, V_ext, Wo):
    B, sq_loc, D = x.shape
    _, skv, hq, dh = K_ext.shape
    Kt = jnp.transpose(K_ext, (0, 2, 3, 1))
    d_out = Wo.shape[1]
    rows = B * sq_loc
    h_grp = hq // N_DEV
    grp_cols = h_grp * dh
    bf16 = jnp.bfloat16
    wq_sh = Wq.shape

    def body(x_ref, wq_ref, k_ref, v_ref, wo_ref, out_ref,
             wqb, wob, pair_send, pair_l, pair_r, fwd_wq, fwd_wo,
             k_stage, k_bf, v_blk, ctx_blk,
             k_sem,
             s_pair_l, s_pair_r, r_pair_l, r_pair_r,
             s_fwdq, r_fwdq, s_fwdo, r_fwdo):
        my = lax.axis_index("i")
        left = lax.rem(my + N_DEV - 1, N_DEV)
        right = lax.rem(my + 1, N_DEV)
        is_even = lax.rem(my, 2) == 0
        MESH = pl.DeviceIdType.MESH

        kdma = pltpu.make_async_copy(k_ref, k_stage, k_sem)
        kdma.start()

        wqb[...] = wq_ref[...].astype(bf16)
        wob[...] = wo_ref[...].astype(bf16)
        pair_send[0] = wqb[...]
        pair_send[1] = wob[...].reshape(wq_sh)
        x2b = x_ref[...].reshape(rows, D).astype(bf16)

        def stage_kv():
            kdma.wait()
            for b in range(B):
                k_bf[:, :, b * skv:(b + 1) * skv] = k_stage[b].astype(bf16)
            v2 = v_ref[...].reshape(B * skv, hq * dh).astype(bf16)
            for j in range(N_DEV):
                v_blk[j] = v2[:, j * grp_cols:(j + 1) * grp_cols]

        barrier_sem = pltpu.get_barrier_semaphore()
        for nbr in (left, right):
            pl.semaphore_signal(barrier_sem, inc=1, device_id=(nbr,),
                                device_id_type=MESH)
        pl.semaphore_wait(barrier_sem, 2)

        def copy(src, dst, ssem, rsem, dev):
            return pltpu.make_async_remote_copy(
                src_ref=src, dst_ref=dst, send_sem=ssem, recv_sem=rsem,
                device_id=(dev,), device_id_type=MESH)

        ri = lax.broadcasted_iota(jnp.int32, (rows, B * skv), 0)
        ci = lax.broadcasted_iota(jnp.int32, (rows, B * skv), 1)
        qb = my * (sq_loc // 64) + lax.rem(ri, sq_loc) // 64
        kb = lax.rem(ci, skv) // 64
        same_b = (ri // sq_loc) == (ci // skv)
        mask = same_b & ((qb == kb) | (lax.rem(kb, 4) == lax.rem(qb, 4)))
        row_keep = jnp.any(mask, axis=1, keepdims=True)
        neg = jnp.float32(-1e9)

        def group_out(wq_val, wo_val, origin):
            qg = (jnp.dot(x2b, wq_val, preferred_element_type=jnp.float32)
                  * 0.125).astype(bf16)
            for h2 in range(h_grp):
                c0 = h2 * dh
                g = origin * h_grp + h2
                kcat = k_bf[g]
                vc = v_blk[origin][:, c0:c0 + dh]
                qc = qg[:, c0:c0 + dh]
                s = jnp.dot(qc, kcat,
                            preferred_element_type=jnp.float32)
                s = jnp.where(mask, s, neg)
                w = jnp.exp(s)
                ws = jnp.where(row_keep,
                               jnp.sum(w, axis=1, keepdims=True), 1.0)
                w = jnp.where(row_keep, w / ws, 0.0)
                ctx_blk[:, c0:c0 + dh] = jnp.dot(
                    w.astype(bf16), vc,
                    preferred_element_type=jnp.float32).astype(bf16)
            return jnp.dot(ctx_blk[...], wo_val,
                           preferred_element_type=jnp.float32)

        @pl.when(is_even)
        def _():
            cq = copy(wqb, fwd_wq, s_fwdq, r_fwdq, left)
            co = copy(wob, fwd_wo, s_fwdo, r_fwdo, right)
            cq.start()
            co.start()

            stage_kv()

            acc = group_out(wqb[...], wob[...], my)

            copy(pair_send, pair_l, s_pair_l, r_pair_l, left).wait_recv()
            acc = acc + group_out(pair_l[0],
                                  pair_l[1].reshape(grp_cols, d_out),
                                  lax.rem(my + N_DEV - 1, N_DEV))
            copy(pair_send, pair_r, s_pair_r, r_pair_r, right).wait_recv()
            acc = acc + group_out(pair_r[0],
                                  pair_r[1].reshape(grp_cols, d_out),
                                  lax.rem(my + 1, N_DEV))

            cq.wait_recv()
            co.wait_recv()
            acc = acc + group_out(fwd_wq[...], fwd_wo[...],
                                  lax.rem(my + 2, N_DEV))

            cq.wait_send()
            co.wait_send()
            out_ref[...] = acc.reshape(B, sq_loc, d_out)

        @pl.when(jnp.logical_not(is_even))
        def _():
            cl = copy(pair_send, pair_r, s_pair_l, r_pair_r, left)
            cr = copy(pair_send, pair_l, s_pair_r, r_pair_l, right)
            cl.start()
            cr.start()

            kdma.wait()

            copy(wqb, fwd_wq, s_fwdq, r_fwdq, right).wait_recv()
            fq = copy(fwd_wq, fwd_wq, s_fwdq, r_fwdq, left)
            fq.start()
            copy(wob, fwd_wo, s_fwdo, r_fwdo, left).wait_recv()
            fo = copy(fwd_wo, fwd_wo, s_fwdo, r_fwdo, right)
            fo.start()

            cl.wait_send()
            cr.wait_send()
            fq.wait_send()
            fo.wait_send()
            out_ref[...] = jnp.zeros((B, sq_loc, d_out), jnp.float32)

    return pl.pallas_call(
        body,
        out_shape=jax.ShapeDtypeStruct((B, sq_loc, d_out), jnp.float32),
        in_specs=[
            pl.BlockSpec(memory_space=pltpu.VMEM),
            pl.BlockSpec(memory_space=pltpu.VMEM),
            pl.BlockSpec(memory_space=pl.ANY),
            pl.BlockSpec(memory_space=pltpu.VMEM),
            pl.BlockSpec(memory_space=pltpu.VMEM),
        ],
        out_specs=pl.BlockSpec(memory_space=pltpu.VMEM),
        scratch_shapes=[
            pltpu.VMEM(wq_sh, bf16),
            pltpu.VMEM(Wo.shape, bf16),
            pltpu.VMEM((2,) + wq_sh, bf16),
            pltpu.VMEM((2,) + wq_sh, bf16),
            pltpu.VMEM((2,) + wq_sh, bf16),
            pltpu.VMEM(wq_sh, bf16),
            pltpu.VMEM(Wo.shape, bf16),
            pltpu.VMEM(Kt.shape, Kt.dtype),
            pltpu.VMEM((hq, dh, B * skv), bf16),
            pltpu.VMEM((N_DEV, B * skv, grp_cols), bf16),
            pltpu.VMEM((rows, grp_cols), bf16),
            pltpu.SemaphoreType.DMA,
            pltpu.SemaphoreType.DMA,
            pltpu.SemaphoreType.DMA,
            pltpu.SemaphoreType.DMA,
            pltpu.SemaphoreType.DMA,
            pltpu.SemaphoreType.DMA,
            pltpu.SemaphoreType.DMA,
            pltpu.SemaphoreType.DMA,
            pltpu.SemaphoreType.DMA,
        ],
        compiler_params=_CompilerParams(collective_id=0),
    )(x, Wq, Kt, V_ext, Wo)
